# baseline (speedup 1.0000x reference)
import jax
import jax.numpy as jnp
from jax.experimental import pallas as pl
from jax.experimental.pallas import tpu as pltpu

_BM = 256


def _xw_kernel(x_ref, w_ref, b_ref, o_ref):
    o_ref[...] = (
        jnp.dot(
            x_ref[...].astype(jnp.bfloat16),
            w_ref[...].astype(jnp.bfloat16),
            preferred_element_type=jnp.float32,
        )
        + b_ref[...]
    )


def _layer1_kernel(a_ref, h_ref, w_ref, b_ref, o_ref, abf_ref):
    a_bf = a_ref[...].astype(jnp.bfloat16)
    abf_ref[...] = a_bf
    acc = jnp.dot(a_bf, h_ref[...].astype(jnp.bfloat16), preferred_element_type=jnp.float32)
    acc = jnp.maximum(acc, 0.0).astype(jnp.bfloat16)
    o_ref[...] = (
        jnp.dot(acc, w_ref[...].astype(jnp.bfloat16), preferred_element_type=jnp.float32)
        + b_ref[...]
    )


def _mid_kernel(a_ref, h_ref, w_ref, b_ref, o_ref):
    acc = jnp.dot(a_ref[...], h_ref[...].astype(jnp.bfloat16), preferred_element_type=jnp.float32)
    acc = jnp.maximum(acc, 0.0).astype(jnp.bfloat16)
    o_ref[...] = (
        jnp.dot(acc, w_ref[...].astype(jnp.bfloat16), preferred_element_type=jnp.float32)
        + b_ref[...]
    )


def _final_kernel(a_ref, h_ref, o_ref):
    o_ref[...] = jnp.dot(
        a_ref[...], h_ref[...].astype(jnp.bfloat16), preferred_element_type=jnp.float32
    )


def kernel(x, adj_t, W1, b1, W2, b2, W3, b3):
    n, d_in = x.shape
    d_hid = W1.shape[1]
    d_out = W3.shape[1]
    bm = min(_BM, n)
    grid = (pl.cdiv(n, bm),)

    b1r = b1.reshape(1, -1)
    b2r = b2.reshape(1, -1)
    b3r = b3.reshape(1, -1)

    p1 = pl.pallas_call(
        _xw_kernel,
        grid=grid,
        in_specs=[
            pl.BlockSpec((bm, d_in), lambda i: (i, 0)),
            pl.BlockSpec((d_in, d_hid), lambda i: (0, 0)),
            pl.BlockSpec((1, d_hid), lambda i: (0, 0)),
        ],
        out_specs=pl.BlockSpec((bm, d_hid), lambda i: (i, 0)),
        out_shape=jax.ShapeDtypeStruct((n, d_hid), jnp.float32),
    )(x, W1, b1r)

    h2, a_bf = pl.pallas_call(
        _layer1_kernel,
        grid=grid,
        in_specs=[
            pl.BlockSpec((bm, n), lambda i: (i, 0)),
            pl.BlockSpec((n, d_hid), lambda i: (0, 0)),
            pl.BlockSpec((d_hid, d_hid), lambda i: (0, 0)),
            pl.BlockSpec((1, d_hid), lambda i: (0, 0)),
        ],
        out_specs=[
            pl.BlockSpec((bm, d_hid), lambda i: (i, 0)),
            pl.BlockSpec((bm, n), lambda i: (i, 0)),
        ],
        out_shape=[
            jax.ShapeDtypeStruct((n, d_hid), jnp.float32),
            jax.ShapeDtypeStruct((n, n), jnp.bfloat16),
        ],
    )(adj_t, p1, W2, b2r)

    h3 = pl.pallas_call(
        _mid_kernel,
        grid=grid,
        in_specs=[
            pl.BlockSpec((bm, n), lambda i: (i, 0)),
            pl.BlockSpec((n, d_hid), lambda i: (0, 0)),
            pl.BlockSpec((d_hid, d_out), lambda i: (0, 0)),
            pl.BlockSpec((1, d_out), lambda i: (0, 0)),
        ],
        out_specs=pl.BlockSpec((bm, d_out), lambda i: (i, 0)),
        out_shape=jax.ShapeDtypeStruct((n, d_out), jnp.float32),
    )(a_bf, h2, W3, b3r)

    out = pl.pallas_call(
        _final_kernel,
        grid=grid,
        in_specs=[
            pl.BlockSpec((bm, n), lambda i: (i, 0)),
            pl.BlockSpec((n, d_out), lambda i: (0, 0)),
        ],
        out_specs=pl.BlockSpec((bm, d_out), lambda i: (i, 0)),
        out_shape=jax.ShapeDtypeStruct((n, d_out), jnp.float32),
    )(a_bf, h3)

    return out
